# Initial kernel scaffold; baseline (speedup 1.0000x reference)
import jax
import jax.numpy as jnp
from jax.experimental import pallas as pl


def kernel(adj, features, w1, b1, w2, b2):
    raise NotImplementedError("write your pallas kernel here")


# speedup vs baseline: 1.8348x; 1.8348x over previous
import functools

import jax
import jax.numpy as jnp
from jax import lax
from jax.experimental import pallas as pl
from jax.experimental.pallas import tpu as pltpu


def _deg_kernel(a_ref, d_ref):
    d_ref[...] = lax.rsqrt(
        1.0 + jnp.sum(a_ref[...], axis=1, keepdims=True))


def _degree_rsqrt(a, t):
    n = a.shape[0]
    return pl.pallas_call(
        _deg_kernel,
        out_shape=jax.ShapeDtypeStruct((n, 1), jnp.float32),
        grid_spec=pltpu.PrefetchScalarGridSpec(
            num_scalar_prefetch=0,
            grid=(n // t,),
            in_specs=[pl.BlockSpec((t, n), lambda i: (i, 0))],
            out_specs=pl.BlockSpec((t, 1), lambda i: (i, 0)),
        ),
        compiler_params=pltpu.CompilerParams(
            dimension_semantics=("parallel",)),
    )(a)


def _fused_kernel(a_ref, x_i_ref, x_k_ref, d_i_ref, d_k_ref,
                  w1_ref, b1_ref, w2_ref, p_ref, acc_ref, accv_ref):
    k = pl.program_id(1)

    @pl.when(k == 0)
    def _():
        acc_ref[...] = d_i_ref[...] * x_i_ref[...]
        accv_ref[...] = d_i_ref[...]

    a_bf = a_ref[...].astype(jnp.bfloat16)
    xs_k = (d_k_ref[...] * x_k_ref[...]).astype(jnp.bfloat16)
    acc_ref[...] += jnp.dot(a_bf, xs_k, preferred_element_type=jnp.float32)
    accv_ref[...] += jnp.dot(a_bf, d_k_ref[...].astype(jnp.bfloat16),
                             preferred_element_type=jnp.float32)

    @pl.when(k == pl.num_programs(1) - 1)
    def _():
        d_i = d_i_ref[...]
        h = jnp.dot(d_i * acc_ref[...], w1_ref[...],
                    preferred_element_type=jnp.float32) + b1_ref[...]
        h = jnp.maximum(h, 0.0)
        m = jnp.dot(d_i * h, w2_ref[...], preferred_element_type=jnp.float32)
        p_ref[0] = lax.dot_general(
            accv_ref[...], m, (((0,), (0,)), ((), ())),
            preferred_element_type=jnp.float32)


def _fused_pass(a, x, d, w1, b1, w2_pad, t):
    n, f_in = x.shape
    h_feats = w1.shape[1]
    c_pad = w2_pad.shape[1]
    return pl.pallas_call(
        _fused_kernel,
        out_shape=jax.ShapeDtypeStruct((n // t, 1, c_pad), jnp.float32),
        grid_spec=pltpu.PrefetchScalarGridSpec(
            num_scalar_prefetch=0,
            grid=(n // t, n // t),
            in_specs=[
                pl.BlockSpec((t, t), lambda i, k: (i, k)),
                pl.BlockSpec((t, f_in), lambda i, k: (i, 0)),
                pl.BlockSpec((t, f_in), lambda i, k: (k, 0)),
                pl.BlockSpec((t, 1), lambda i, k: (i, 0)),
                pl.BlockSpec((t, 1), lambda i, k: (k, 0)),
                pl.BlockSpec((f_in, h_feats), lambda i, k: (0, 0)),
                pl.BlockSpec((1, h_feats), lambda i, k: (0, 0)),
                pl.BlockSpec((h_feats, c_pad), lambda i, k: (0, 0)),
            ],
            out_specs=pl.BlockSpec((1, 1, c_pad), lambda i, k: (i, 0, 0)),
            scratch_shapes=[
                pltpu.VMEM((t, f_in), jnp.float32),
                pltpu.VMEM((t, 1), jnp.float32),
            ],
        ),
        compiler_params=pltpu.CompilerParams(
            dimension_semantics=("parallel", "arbitrary")),
    )(a, x, x, d, d, w1, b1, w2_pad)


def _finalize_kernel(p_ref, b2_ref, out_ref, *, inv_n):
    out_ref[...] = jnp.sum(p_ref[...], axis=0) * inv_n + b2_ref[...]


def _finalize(partials, b2_pad, n):
    c_pad = partials.shape[-1]
    return pl.pallas_call(
        functools.partial(_finalize_kernel, inv_n=1.0 / n),
        out_shape=jax.ShapeDtypeStruct((1, c_pad), jnp.float32),
    )(partials, b2_pad)


def kernel(adj, features, w1, b1, w2, b2):
    n = adj.shape[0]
    num_classes = w2.shape[1]
    c_pad = max(128, ((num_classes + 127) // 128) * 128)
    t = 512 if n % 512 == 0 else n

    w2_pad = jnp.pad(w2, ((0, 0), (0, c_pad - num_classes)))
    b2_pad = jnp.pad(b2, ((0, 0), (0, c_pad - num_classes)))

    d = _degree_rsqrt(adj, t)
    partials = _fused_pass(adj, features, d, w1, b1, w2_pad, t)
    out_pad = _finalize(partials, b2_pad, n)
    return out_pad[:, :num_classes]


# speedup vs baseline: 1.9804x; 1.0794x over previous
import functools

import jax
import jax.numpy as jnp
from jax import lax
from jax.experimental import pallas as pl
from jax.experimental.pallas import tpu as pltpu


def _prep_kernel(a_ref, x_ref, d_ref, abf_ref, xs_ref, vp_ref):
    a = a_ref[...]
    d = lax.rsqrt(1.0 + jnp.sum(a, axis=1, keepdims=True))
    d_ref[...] = d
    abf_ref[...] = a.astype(jnp.bfloat16)
    xs_ref[...] = (d * x_ref[...]).astype(jnp.bfloat16)
    vp_ref[0] = lax.dot_general(d, a, (((0,), (0,)), ((), ())),
                                preferred_element_type=jnp.float32)


def _prep_pass(a, x, t):
    n, f_in = x.shape
    return pl.pallas_call(
        _prep_kernel,
        out_shape=(
            jax.ShapeDtypeStruct((n, 1), jnp.float32),
            jax.ShapeDtypeStruct((n, n), jnp.bfloat16),
            jax.ShapeDtypeStruct((n, f_in), jnp.bfloat16),
            jax.ShapeDtypeStruct((n // t, 1, n), jnp.float32),
        ),
        grid_spec=pltpu.PrefetchScalarGridSpec(
            num_scalar_prefetch=0,
            grid=(n // t,),
            in_specs=[
                pl.BlockSpec((t, n), lambda i: (i, 0)),
                pl.BlockSpec((t, f_in), lambda i: (i, 0)),
            ],
            out_specs=(
                pl.BlockSpec((t, 1), lambda i: (i, 0)),
                pl.BlockSpec((t, n), lambda i: (i, 0)),
                pl.BlockSpec((t, f_in), lambda i: (i, 0)),
                pl.BlockSpec((1, 1, n), lambda i: (i, 0, 0)),
            ),
        ),
        compiler_params=pltpu.CompilerParams(
            dimension_semantics=("parallel",)),
    )(a, x)


def _fused_kernel(a_ref, x_i_ref, xs_k_ref, d_i_ref, vp_ref,
                  w1_ref, b1_ref, w2_ref, p_ref, acc_ref):
    k = pl.program_id(1)

    @pl.when(k == 0)
    def _():
        acc_ref[...] = d_i_ref[...] * x_i_ref[...]

    acc_ref[...] += jnp.dot(a_ref[...], xs_k_ref[...],
                            preferred_element_type=jnp.float32)

    @pl.when(k == pl.num_programs(1) - 1)
    def _():
        d_i = d_i_ref[...]
        h = jnp.dot(d_i * acc_ref[...], w1_ref[...],
                    preferred_element_type=jnp.float32) + b1_ref[...]
        h = jnp.maximum(h, 0.0)
        m = jnp.dot(d_i * h, w2_ref[...], preferred_element_type=jnp.float32)
        va = jnp.sum(vp_ref[...], axis=0)
        p_ref[0] = (
            jnp.dot(va, m, preferred_element_type=jnp.float32)
            + lax.dot_general(d_i, m, (((0,), (0,)), ((), ())),
                              preferred_element_type=jnp.float32))


def _fused_pass(a_bf, x, xs_bf, d, vparts, w1, b1, w2, t):
    n, f_in = x.shape
    h_feats = w1.shape[1]
    c = w2.shape[1]
    s = n // t
    return pl.pallas_call(
        _fused_kernel,
        out_shape=jax.ShapeDtypeStruct((s, 1, c), jnp.float32),
        grid_spec=pltpu.PrefetchScalarGridSpec(
            num_scalar_prefetch=0,
            grid=(s, s),
            in_specs=[
                pl.BlockSpec((t, t), lambda i, k: (i, k)),
                pl.BlockSpec((t, f_in), lambda i, k: (i, 0)),
                pl.BlockSpec((t, f_in), lambda i, k: (k, 0)),
                pl.BlockSpec((t, 1), lambda i, k: (i, 0)),
                pl.BlockSpec((s, 1, t), lambda i, k: (0, 0, i)),
                pl.BlockSpec((f_in, h_feats), lambda i, k: (0, 0)),
                pl.BlockSpec((1, h_feats), lambda i, k: (0, 0)),
                pl.BlockSpec((h_feats, c), lambda i, k: (0, 0)),
            ],
            out_specs=pl.BlockSpec((1, 1, c), lambda i, k: (i, 0, 0)),
            scratch_shapes=[pltpu.VMEM((t, f_in), jnp.float32)],
        ),
        compiler_params=pltpu.CompilerParams(
            dimension_semantics=("parallel", "arbitrary")),
    )(a_bf, x, xs_bf, d, vparts, w1, b1, w2)


def _finalize_kernel(p_ref, b2_ref, out_ref, *, inv_n):
    out_ref[...] = jnp.sum(p_ref[...], axis=0) * inv_n + b2_ref[...]


def _finalize(partials, b2, n):
    c = partials.shape[-1]
    return pl.pallas_call(
        functools.partial(_finalize_kernel, inv_n=1.0 / n),
        out_shape=jax.ShapeDtypeStruct((1, c), jnp.float32),
    )(partials, b2)


def kernel(adj, features, w1, b1, w2, b2):
    n = adj.shape[0]
    t = 512 if n % 512 == 0 else n

    d, a_bf, xs_bf, vparts = _prep_pass(adj, features, t)
    partials = _fused_pass(a_bf, features, xs_bf, d, vparts, w1, b1, w2, t)
    return _finalize(partials, b2, n)


# speedup vs baseline: 5.0789x; 2.5646x over previous
import functools

import jax
import jax.numpy as jnp
from jax import lax
from jax.experimental import pallas as pl
from jax.experimental.pallas import tpu as pltpu


def _mono_kernel(a_ref, x_ref, w1_ref, b1_ref, w2_ref, b2_ref, out_ref,
                 cache_ref, xs_ref, dbf_ref, d_ref, acc_ref, v_ref,
                 *, s, t, inv_n):
    k = pl.program_id(0)
    f32 = jnp.float32

    @pl.when(k == 0)
    def _():
        acc_ref[...] = jnp.zeros_like(acc_ref)
        v_ref[...] = jnp.zeros_like(v_ref)

    def strip_dots():
        xs = xs_ref[...]
        dbf = dbf_ref[...]
        for j in range(s):
            cj = cache_ref[:, j * t:(j + 1) * t]
            acc_ref[j * t:(j + 1) * t, :] += lax.dot_general(
                cj, xs, (((0,), (0,)), ((), ())), preferred_element_type=f32)
            v_ref[:, j * t:(j + 1) * t] += lax.dot_general(
                dbf, cj, (((0,), (0,)), ((), ())), preferred_element_type=f32)

    @pl.when(k > 0)
    def _():
        strip_dots()

    rs = jnp.zeros((t, 1), f32)
    for j in range(s):
        chunk = a_ref[:, j * t:(j + 1) * t]
        rs = rs + jnp.sum(chunk, axis=1, keepdims=True)
        cache_ref[:, j * t:(j + 1) * t] = chunk.astype(jnp.bfloat16)
    d = lax.rsqrt(1.0 + rs)
    d_ref[pl.ds(k * t, t), :] = d
    xs_f = d * x_ref[...]
    acc_ref[pl.ds(k * t, t), :] += xs_f
    xs_ref[...] = xs_f.astype(jnp.bfloat16)
    dbf_ref[...] = d.astype(jnp.bfloat16)

    @pl.when(k == s - 1)
    def _():
        strip_dots()
        p = jnp.zeros_like(out_ref)
        for i in range(s):
            d_i = d_ref[i * t:(i + 1) * t, :]
            h = jnp.dot(d_i * acc_ref[i * t:(i + 1) * t, :], w1_ref[...],
                        preferred_element_type=f32) + b1_ref[...]
            h = jnp.maximum(h, 0.0)
            m = jnp.dot(d_i * h, w2_ref[...], preferred_element_type=f32)
            p = (p + jnp.dot(v_ref[:, i * t:(i + 1) * t], m,
                             preferred_element_type=f32)
                 + lax.dot_general(d_i, m, (((0,), (0,)), ((), ())),
                                   preferred_element_type=f32))
        out_ref[...] = p * inv_n + b2_ref[...]


def _mono_pass(a, x, w1, b1, w2, b2, t):
    n, f_in = x.shape
    h_feats = w1.shape[1]
    c = w2.shape[1]
    s = n // t
    body = functools.partial(_mono_kernel, s=s, t=t, inv_n=1.0 / n)
    return pl.pallas_call(
        body,
        out_shape=jax.ShapeDtypeStruct((1, c), jnp.float32),
        grid_spec=pltpu.PrefetchScalarGridSpec(
            num_scalar_prefetch=0,
            grid=(s,),
            in_specs=[
                pl.BlockSpec((t, n), lambda k: (k, 0)),
                pl.BlockSpec((t, f_in), lambda k: (k, 0)),
                pl.BlockSpec((f_in, h_feats), lambda k: (0, 0)),
                pl.BlockSpec((1, h_feats), lambda k: (0, 0)),
                pl.BlockSpec((h_feats, c), lambda k: (0, 0)),
                pl.BlockSpec((1, c), lambda k: (0, 0)),
            ],
            out_specs=pl.BlockSpec((1, c), lambda k: (0, 0)),
            scratch_shapes=[
                pltpu.VMEM((t, n), jnp.bfloat16),
                pltpu.VMEM((t, f_in), jnp.bfloat16),
                pltpu.VMEM((t, 1), jnp.bfloat16),
                pltpu.VMEM((n, 1), jnp.float32),
                pltpu.VMEM((n, f_in), jnp.float32),
                pltpu.VMEM((1, n), jnp.float32),
            ],
        ),
        compiler_params=pltpu.CompilerParams(
            dimension_semantics=("arbitrary",)),
    )(a, x, w1, b1, w2, b2)


def kernel(adj, features, w1, b1, w2, b2):
    n = adj.shape[0]
    t = 512 if n % 512 == 0 else n
    return _mono_pass(adj, features, w1, b1, w2, b2, t)


# speedup vs baseline: 5.1810x; 1.0201x over previous
import functools

import jax
import jax.numpy as jnp
from jax import lax
from jax.experimental import pallas as pl
from jax.experimental.pallas import tpu as pltpu


def _mono_kernel(a_ref, x_ref, w1_ref, b1_ref, w2_ref, b2_ref, out_ref,
                 cache_ref, xs_ref, dbf_ref, d_ref, acc_ref, v_ref,
                 *, s, t, inv_n):
    k = pl.program_id(0)
    f32 = jnp.float32

    @pl.when(k == 0)
    def _():
        acc_ref[...] = jnp.zeros_like(acc_ref)
        v_ref[...] = jnp.zeros_like(v_ref)

    def strip_dots():
        xs = xs_ref[...]
        dbf = dbf_ref[...]
        for j in range(s):
            cj = cache_ref[:, j * t:(j + 1) * t]
            acc_ref[j * t:(j + 1) * t, :] += lax.dot_general(
                cj, xs, (((0,), (0,)), ((), ())), preferred_element_type=f32)
            v_ref[:, j * t:(j + 1) * t] += lax.dot_general(
                dbf, cj, (((0,), (0,)), ((), ())), preferred_element_type=f32)

    @pl.when(k > 0)
    def _():
        strip_dots()

    rs = jnp.zeros((t, 1), f32)
    for j in range(s):
        chunk = a_ref[:, j * t:(j + 1) * t]
        rs = rs + jnp.sum(chunk, axis=1, keepdims=True)
        cache_ref[:, j * t:(j + 1) * t] = chunk.astype(jnp.bfloat16)
    d = lax.rsqrt(1.0 + rs)
    d_ref[pl.ds(k * t, t), :] = d
    xs_f = d * x_ref[...]
    acc_ref[pl.ds(k * t, t), :] += xs_f
    xs_ref[...] = xs_f.astype(jnp.bfloat16)
    dbf_ref[...] = d.astype(jnp.bfloat16)

    @pl.when(k == s - 1)
    def _():
        strip_dots()
        p = jnp.zeros_like(out_ref)
        for i in range(s):
            d_i = d_ref[i * t:(i + 1) * t, :]
            h = jnp.dot(d_i * acc_ref[i * t:(i + 1) * t, :], w1_ref[...],
                        preferred_element_type=f32) + b1_ref[...]
            h = jnp.maximum(h, 0.0)
            m = jnp.dot(d_i * h, w2_ref[...], preferred_element_type=f32)
            p = (p + jnp.dot(v_ref[:, i * t:(i + 1) * t], m,
                             preferred_element_type=f32)
                 + lax.dot_general(d_i, m, (((0,), (0,)), ((), ())),
                                   preferred_element_type=f32))
        out_ref[...] = p * inv_n + b2_ref[...]


def _mono_pass(a, x, w1, b1, w2, b2, t):
    n, f_in = x.shape
    h_feats = w1.shape[1]
    c = w2.shape[1]
    s = n // t
    body = functools.partial(_mono_kernel, s=s, t=t, inv_n=1.0 / n)
    return pl.pallas_call(
        body,
        out_shape=jax.ShapeDtypeStruct((1, c), jnp.float32),
        grid_spec=pltpu.PrefetchScalarGridSpec(
            num_scalar_prefetch=0,
            grid=(s,),
            in_specs=[
                pl.BlockSpec((t, n), lambda k: (k, 0)),
                pl.BlockSpec((t, f_in), lambda k: (k, 0)),
                pl.BlockSpec((f_in, h_feats), lambda k: (0, 0)),
                pl.BlockSpec((1, h_feats), lambda k: (0, 0)),
                pl.BlockSpec((h_feats, c), lambda k: (0, 0)),
                pl.BlockSpec((1, c), lambda k: (0, 0)),
            ],
            out_specs=pl.BlockSpec((1, c), lambda k: (0, 0)),
            scratch_shapes=[
                pltpu.VMEM((t, n), jnp.bfloat16),
                pltpu.VMEM((t, f_in), jnp.bfloat16),
                pltpu.VMEM((t, 1), jnp.bfloat16),
                pltpu.VMEM((n, 1), jnp.float32),
                pltpu.VMEM((n, f_in), jnp.float32),
                pltpu.VMEM((1, n), jnp.float32),
            ],
        ),
        compiler_params=pltpu.CompilerParams(
            dimension_semantics=("arbitrary",)),
    )(a, x, w1, b1, w2, b2)


def kernel(adj, features, w1, b1, w2, b2):
    n = adj.shape[0]
    t = 1024 if n % 1024 == 0 else (512 if n % 512 == 0 else n)
    return _mono_pass(adj, features, w1, b1, w2, b2, t)
